# baseline (speedup 1.0000x reference)
import functools

import jax
import jax.numpy as jnp
from jax import lax
from jax.experimental import pallas as pl
from jax.experimental.pallas import tpu as pltpu
from jax.experimental.pallas import tpu_sc as plsc

DIM = 32
NUM_CORES = 2
NUM_SUBCORES = 16
NW = NUM_CORES * NUM_SUBCORES
CHUNK = 1000
NBUF = 2


def _make_gather(ntok):
    bpw = ntok // NW
    nchunks = bpw // CHUNK

    @functools.partial(
        pl.kernel,
        mesh=plsc.VectorSubcoreMesh(core_axis_name="c", subcore_axis_name="s"),
        out_type=jax.ShapeDtypeStruct((ntok, DIM), jnp.float32),
        scratch_types=[
            pltpu.VMEM((bpw,), jnp.int32),
            pltpu.VMEM((CHUNK, DIM), jnp.float32),
            pltpu.VMEM((CHUNK, DIM), jnp.float32),
            pltpu.SemaphoreType.DMA,
            pltpu.SemaphoreType.DMA,
            pltpu.SemaphoreType.DMA,
            pltpu.SemaphoreType.DMA,
        ],
        compiler_params=pltpu.CompilerParams(use_tc_tiling_on_sc=False),
    )
    def _sc_gather(tok_hbm, w_hbm, out_hbm, idx_v, rows0, rows1, sg0, sg1,
                   sw0, sw1):
        rows = (rows0, rows1)
        sem_g = (sg0, sg1)
        sem_w = (sw0, sw1)
        wid = lax.axis_index("s") * NUM_CORES + lax.axis_index("c")
        base = wid * bpw

        pltpu.sync_copy(tok_hbm.at[pl.ds(base, bpw)], idx_v)

        def pair(gp, carry):
            for b in range(NBUF):
                g = gp * NBUF + b
                @pl.when(g >= NBUF)
                def _():
                    pltpu.make_async_copy(
                        rows[b], out_hbm.at[pl.ds(base, CHUNK)], sem_w[b]
                    ).wait()

                pltpu.async_copy(
                    w_hbm.at[idx_v.at[pl.ds(g * CHUNK, CHUNK)]], rows[b],
                    sem_g[b],
                ).wait()
                pltpu.async_copy(
                    rows[b], out_hbm.at[pl.ds(base + g * CHUNK, CHUNK)],
                    sem_w[b],
                )
            return carry

        lax.fori_loop(0, nchunks // NBUF, pair, 0)
        for b in range(NBUF):
            pltpu.make_async_copy(
                rows[b], out_hbm.at[pl.ds(base, CHUNK)], sem_w[b]
            ).wait()

    return _sc_gather


_gather_half = _make_gather(1024 * 25 * 20)


def kernel(tokens, weight):
    outs = []
    for h in range(2):
        tokh = tokens[:, h * 25 : (h + 1) * 25, :].reshape(-1)
        out = _gather_half(tokh.astype(jnp.int32), weight)
        outs.append(out.reshape(1024, 25, 20, DIM))
    return jnp.concatenate(outs, axis=1)
